# baseline (speedup 1.0000x reference)
import functools

import jax
import jax.numpy as jnp
from jax import lax
from jax.experimental import pallas as pl
from jax.experimental.pallas import tpu as pltpu
from jax.experimental.pallas import tpu_sc as plsc

N_NODES = 10000
N_PAD = 10240
IN_CH, HID_CH, OUT_CH = 128, 64, 64
K = 10
ALPHA = 0.1

NC, NS = 2, 16
NW = NC * NS
NB = NW
BW = N_PAD // NB
CHUNK = 128
NCH = 82
E_PER_W = NCH * CHUNK
E_PAD = NW * E_PER_W
ES_TOTAL = E_PAD + NB * CHUNK
SUP = 84
ES_ALLOC = ES_TOTAL + SUP * CHUNK
FB = BW * OUT_CH

_mesh = plsc.VectorSubcoreMesh(
    core_axis_name="c", subcore_axis_name="s", num_cores=NC, num_subcores=NS)
_sc_params = pltpu.CompilerParams(
    needs_layout_passes=False, use_tc_tiling_on_sc=False)


@functools.partial(
    pl.kernel,
    out_type=jax.ShapeDtypeStruct((NC, N_PAD), jnp.float32),
    mesh=_mesh,
    compiler_params=_sc_params,
    scratch_types=[
        pltpu.VMEM((NCH, CHUNK), jnp.int32),
        pltpu.VMEM((NCH, CHUNK), jnp.float32),
        pltpu.VMEM((N_PAD,), jnp.float32),
        pltpu.VMEM_SHARED((N_PAD,), jnp.float32),
    ],
)
def _deg_kernel(col_hbm, w_hbm, degp_hbm, col_v, w_v, bounce_v, deg_sh):
    cid = lax.axis_index("c")
    sid = lax.axis_index("s")
    wid = sid * NC + cid
    pltpu.sync_copy(col_hbm.at[wid], col_v)
    pltpu.sync_copy(w_hbm.at[wid], w_v)

    zero16 = jnp.zeros((16,), jnp.float32)

    def _zero(i, carry):
        bounce_v[pl.ds(i * 16, 16)] = zero16
        return carry

    lax.fori_loop(0, N_PAD // 16, _zero, 0)

    @pl.when(sid == 0)
    def _():
        pltpu.sync_copy(bounce_v, deg_sh)

    plsc.subcore_barrier()

    def _scatter(j, carry):
        pltpu.sync_copy(w_v.at[j], deg_sh.at[col_v.at[j]], add=True)
        return carry

    lax.fori_loop(0, NCH, _scatter, 0)
    plsc.subcore_barrier()

    @pl.when(sid == 0)
    def _():
        pltpu.sync_copy(deg_sh, bounce_v)
        pltpu.sync_copy(bounce_v, degp_hbm.at[cid])


@functools.partial(
    pl.kernel,
    out_type=jax.ShapeDtypeStruct((NW, NB), jnp.int32),
    mesh=_mesh,
    compiler_params=_sc_params,
    scratch_types=[
        pltpu.VMEM((E_PER_W,), jnp.int32),
        pltpu.VMEM((NB,), jnp.int32),
    ],
)
def _count_kernel(colf, counts_hbm, col_v, cnt_v):
    cid = lax.axis_index("c")
    sid = lax.axis_index("s")
    wid = sid * NC + cid
    pltpu.sync_copy(colf.at[wid], col_v)
    z16 = jnp.zeros((16,), jnp.int32)
    cnt_v[pl.ds(0, 16)] = z16
    cnt_v[pl.ds(16, 16)] = z16

    def _body(g, carry):
        c16 = col_v[pl.ds(g * 16, 16)]
        b16 = lax.div(c16, BW)
        rank, lastm = plsc.scan_count(b16)
        plsc.addupdate_scatter(cnt_v, [b16], rank, mask=lastm)
        return carry

    lax.fori_loop(0, E_PER_W // 16, _body, 0)
    pltpu.sync_copy(cnt_v, counts_hbm.at[wid])


def _offsets_body(cnt_ref, off_ref, meta_ref):
    c = cnt_ref[...].astype(jnp.float32)
    ti = lax.broadcasted_iota(jnp.int32, (NW, NW), 0)
    tj = lax.broadcasted_iota(jnp.int32, (NW, NW), 1)
    lower = (ti > tj).astype(jnp.float32)
    colcum = jnp.dot(lower, c, preferred_element_type=jnp.float32,
                     precision=lax.Precision.HIGHEST)
    tot = jnp.sum(c, axis=0)
    atot = jnp.ceil(tot / CHUNK) * CHUNK
    astart = jnp.dot(lower, atot[:, None],
                     preferred_element_type=jnp.float32,
                     precision=lax.Precision.HIGHEST)[:, 0]
    off_ref[...] = (astart[None, :] + colcum).astype(jnp.int32)
    meta = jnp.concatenate(
        [astart[:, None], atot[:, None], tot[:, None],
         jnp.zeros((NB, 5), jnp.float32)], axis=1)
    meta_ref[...] = meta.astype(jnp.int32)


def _offsets(counts):
    return pl.pallas_call(
        _offsets_body,
        out_shape=(
            jax.ShapeDtypeStruct((NW, NB), jnp.int32),
            jax.ShapeDtypeStruct((NB, 8), jnp.int32),
        ),
    )(counts)


@functools.partial(
    pl.kernel,
    out_type=(
        jax.ShapeDtypeStruct((ES_ALLOC,), jnp.int32),
        jax.ShapeDtypeStruct((ES_ALLOC,), jnp.int32),
        jax.ShapeDtypeStruct((ES_ALLOC,), jnp.float32),
    ),
    mesh=_mesh,
    compiler_params=_sc_params,
    scratch_types=[
        pltpu.VMEM((E_PER_W,), jnp.int32),
        pltpu.VMEM((E_PER_W,), jnp.int32),
        pltpu.VMEM((E_PER_W,), jnp.float32),
        pltpu.VMEM((NB,), jnp.int32),
        pltpu.VMEM((CHUNK,), jnp.int32),
        pltpu.SemaphoreType.DMA,
    ],
)
def _part_kernel(rowf, colf, wf, off_hbm, rowp, colp, wp,
                 row_v, col_v, w_v, offl_v, slotb_v, sem):
    cid = lax.axis_index("c")
    sid = lax.axis_index("s")
    wid = sid * NC + cid
    pltpu.sync_copy(rowf.at[wid], row_v)
    pltpu.sync_copy(colf.at[wid], col_v)
    pltpu.sync_copy(wf.at[wid], w_v)
    pltpu.sync_copy(off_hbm.at[wid], offl_v)

    def _blk(k, carry):
        def _g(g2, carry2):
            c16 = col_v[pl.ds(k * CHUNK + g2 * 16, 16)]
            b16 = lax.div(c16, BW)
            rank, lastm = plsc.scan_count(b16)
            base16 = plsc.load_gather(offl_v, [b16])
            slotb_v[pl.ds(g2 * 16, 16)] = base16 + rank - 1
            plsc.addupdate_scatter(offl_v, [b16], rank, mask=lastm)
            return carry2

        lax.fori_loop(0, CHUNK // 16, _g, 0)
        o = pl.multiple_of(k * CHUNK, CHUNK)
        cp0 = pltpu.async_copy(row_v.at[pl.ds(o, CHUNK)],
                               rowp.at[slotb_v], sem)
        cp1 = pltpu.async_copy(col_v.at[pl.ds(o, CHUNK)],
                               colp.at[slotb_v], sem)
        cp2 = pltpu.async_copy(w_v.at[pl.ds(o, CHUNK)],
                               wp.at[slotb_v], sem)
        cp0.wait()
        cp1.wait()
        cp2.wait()
        return carry

    lax.fori_loop(0, NCH, _blk, 0)


ESW = ES_TOTAL // NW


@functools.partial(
    pl.kernel,
    out_type=jax.ShapeDtypeStruct((ES_ALLOC,), jnp.float32),
    mesh=_mesh,
    compiler_params=_sc_params,
    scratch_types=[
        pltpu.VMEM((N_PAD,), jnp.float32),
        pltpu.VMEM((ESW,), jnp.int32),
        pltpu.VMEM((ESW,), jnp.int32),
        pltpu.VMEM((ESW,), jnp.float32),
        pltpu.VMEM((ESW,), jnp.float32),
    ],
)
def _norm_kernel(rowp2, colp2, wp2, dinv_hbm, normf, dinv_v, row_v, col_v,
                 w_v, norm_v):
    cid = lax.axis_index("c")
    sid = lax.axis_index("s")
    wid = sid * NC + cid
    eo = pl.multiple_of(wid * ESW, 8)
    pltpu.sync_copy(dinv_hbm, dinv_v)
    pltpu.sync_copy(rowp2.at[pl.ds(eo, ESW)], row_v)
    pltpu.sync_copy(colp2.at[pl.ds(eo, ESW)], col_v)
    pltpu.sync_copy(wp2.at[pl.ds(eo, ESW)], w_v)

    def _body(g, carry):
        r16 = jnp.clip(row_v[pl.ds(g * 16, 16)], 0, N_PAD - 1)
        c16 = jnp.clip(col_v[pl.ds(g * 16, 16)], 0, N_PAD - 1)
        w16 = w_v[pl.ds(g * 16, 16)]
        dr = plsc.load_gather(dinv_v, [r16])
        dc = plsc.load_gather(dinv_v, [c16])
        norm_v[pl.ds(g * 16, 16)] = dr * w16 * dc
        return carry

    lax.fori_loop(0, ESW // 16, _body, 0)
    pltpu.sync_copy(norm_v, normf.at[pl.ds(eo, ESW)])


@functools.partial(
    pl.kernel,
    out_type=jax.ShapeDtypeStruct((N_PAD * OUT_CH,), jnp.float32),
    mesh=_mesh,
    compiler_params=_sc_params,
    scratch_types=[
        pltpu.VMEM((FB,), jnp.float32),
        pltpu.VMEM((SUP * CHUNK,), jnp.int32),
        pltpu.VMEM((SUP * CHUNK,), jnp.int32),
        pltpu.VMEM((SUP * CHUNK,), jnp.float32),
        pltpu.VMEM((CHUNK, OUT_CH), jnp.float32),
        pltpu.VMEM((CHUNK, OUT_CH), jnp.float32),
        pltpu.VMEM((CHUNK, OUT_CH), jnp.float32),
        pltpu.VMEM((CHUNK, OUT_CH), jnp.float32),
        pltpu.VMEM((16,), jnp.int32),
        pltpu.SemaphoreType.DMA,
        pltpu.SemaphoreType.DMA,
        pltpu.SemaphoreType.DMA,
        pltpu.SemaphoreType.DMA,
    ],
)
def _round_kernel(rowp, colp, normp, meta_hbm, h2d, hout,
                  agg_v, re_v, ce_v, ne_v, gb0, gb1, gb2, gb3,
                  mv, sem0, sem1, sem2, sem3):
    cid = lax.axis_index("c")
    sid = lax.axis_index("s")
    wid = sid * NC + cid
    pltpu.sync_copy(meta_hbm.at[wid], mv.at[pl.ds(0, 8)])
    m16 = mv[pl.ds(0, 16)]
    astart = m16[0]
    atot = m16[1]
    tot = m16[2]
    ntrip = lax.div(atot, CHUNK)
    colbase = wid * BW
    fbo = pl.multiple_of(wid * FB, FB)

    gbs = (gb0, gb1, gb2, gb3)
    sems = (sem0, sem1, sem2, sem3)

    z16 = jnp.zeros((16,), jnp.float32)

    def _zero(i, carry):
        agg_v[pl.ds(i * 16, 16)] = z16
        return carry

    lax.fori_loop(0, FB // 16, _zero, 0)

    lane16 = lax.iota(jnp.int32, 16)
    cf = [f * 16 + lane16 for f in range(OUT_CH // 16)]

    def _accum(jg, jl, gb):
        def _grp(g, carry):
            c16 = ce_v[pl.ds(jl * CHUNK + g * 16, 16)]
            lc16 = jnp.clip(c16 - colbase, 0, BW - 1) * OUT_CH
            n16r = ne_v[pl.ds(jl * CHUNK + g * 16, 16)]
            mask = (jg * CHUNK + g * 16 + lane16) < tot
            n16 = jnp.where(mask, n16r, 0.0)
            for e in range(16):
                sel = jnp.full((16,), e, jnp.int32)
                ne = jnp.take_along_axis(n16, sel, axis=0)
                ab = jnp.take_along_axis(lc16, sel, axis=0)
                r = g * 16 + e
                for f in range(OUT_CH // 16):
                    plsc.addupdate_scatter(
                        agg_v, [ab + cf[f]],
                        gb[r, pl.ds(f * 16, 16)] * ne)
            return carry

        lax.fori_loop(0, CHUNK // 16, _grp, 0)

    nsup = lax.div(ntrip + (SUP - 1), SUP)

    def _sup(s, carry):
        cb = s * SUP
        eo = pl.multiple_of(astart + cb * CHUNK, CHUNK)
        pltpu.sync_copy(rowp.at[pl.ds(eo, SUP * CHUNK)], re_v)
        pltpu.sync_copy(colp.at[pl.ds(eo, SUP * CHUNK)], ce_v)
        pltpu.sync_copy(normp.at[pl.ds(eo, SUP * CHUNK)], ne_v)

        def _san(i, c2):
            re_v[pl.ds(i * 16, 16)] = jnp.clip(
                re_v[pl.ds(i * 16, 16)], 0, N_PAD - 1)
            return c2

        lax.fori_loop(0, SUP * CHUNK // 16, _san, 0)
        nloc = jnp.minimum(SUP, ntrip - cb)

        for b in range(4):
            @pl.when(b < nloc)
            def _(b=b):
                pltpu.async_copy(
                    h2d.at[re_v.at[pl.ds(b * CHUNK, CHUNK)]], gbs[b], sems[b])

        def _quad(q, c2):
            for b in range(4):
                j = q * 4 + b

                @pl.when(j < nloc)
                def _(b=b, j=j):
                    pltpu.make_async_copy(
                        h2d.at[re_v.at[pl.ds(j * CHUNK, CHUNK)]],
                        gbs[b], sems[b]).wait()
                    _accum(cb + j, j, gbs[b])

                    @pl.when(j + 4 < nloc)
                    def _(b=b, j=j):
                        pltpu.async_copy(
                            h2d.at[re_v.at[pl.ds((j + 4) * CHUNK, CHUNK)]],
                            gbs[b], sems[b])

            return c2

        lax.fori_loop(0, lax.div(nloc + 3, 4), _quad, 0)
        return carry

    lax.fori_loop(0, nsup, _sup, 0)
    pltpu.sync_copy(agg_v, hout.at[pl.ds(fbo, FB)])


def _mlp_body(x_ref, w1_ref, b1_ref, w2_ref, b2_ref, o_ref):
    h = jnp.dot(x_ref[...], w1_ref[...], preferred_element_type=jnp.float32)
    h = jnp.maximum(h + b1_ref[...], 0.0)
    h = jnp.dot(h, w2_ref[...], preferred_element_type=jnp.float32)
    o_ref[...] = h + b2_ref[...]


def _mlp(x, W1, b1, W2, b2):
    blk = 1000
    return pl.pallas_call(
        _mlp_body,
        grid=(N_NODES // blk,),
        in_specs=[
            pl.BlockSpec((blk, IN_CH), lambda i: (i, 0)),
            pl.BlockSpec((IN_CH, HID_CH), lambda i: (0, 0)),
            pl.BlockSpec((1, HID_CH), lambda i: (0, 0)),
            pl.BlockSpec((HID_CH, OUT_CH), lambda i: (0, 0)),
            pl.BlockSpec((1, OUT_CH), lambda i: (0, 0)),
        ],
        out_specs=pl.BlockSpec((blk, OUT_CH), lambda i: (i, 0)),
        out_shape=jax.ShapeDtypeStruct((N_NODES, OUT_CH), jnp.float32),
    )(x, W1, b1.reshape(1, HID_CH), W2, b2.reshape(1, OUT_CH))


def _combine_body(agg_ref, h0_ref, o_ref):
    o_ref[...] = (1.0 - ALPHA) * agg_ref[...] + ALPHA * h0_ref[...]


def _combine(agg2d, h0p):
    blk = 1024
    return pl.pallas_call(
        _combine_body,
        grid=(N_PAD // blk,),
        in_specs=[
            pl.BlockSpec((blk, OUT_CH), lambda i: (i, 0)),
            pl.BlockSpec((blk, OUT_CH), lambda i: (i, 0)),
        ],
        out_specs=pl.BlockSpec((blk, OUT_CH), lambda i: (i, 0)),
        out_shape=jax.ShapeDtypeStruct((N_PAD, OUT_CH), jnp.float32),
    )(agg2d, h0p)


def _dinv_body(degp_ref, o_ref):
    deg = degp_ref[0] + degp_ref[1]
    safe = jnp.where(deg > 0, deg, 1.0)
    o_ref[...] = jnp.where(deg > 0, lax.rsqrt(safe), 0.0)


def _dinv(degp):
    return pl.pallas_call(
        _dinv_body,
        out_shape=jax.ShapeDtypeStruct((N_PAD // 128, 128), jnp.float32),
    )(degp.reshape(NC, N_PAD // 128, 128))


def kernel(x, edge_index, edge_weight, W1, b1, W2, b2):
    pad = E_PAD - (edge_index.shape[1] + N_NODES)
    loop = jnp.arange(N_NODES, dtype=jnp.int32)
    zpad_i = jnp.zeros((pad,), jnp.int32)
    row = jnp.concatenate([edge_index[0].astype(jnp.int32), loop, zpad_i])
    col = jnp.concatenate([edge_index[1].astype(jnp.int32), loop, zpad_i])
    w = jnp.concatenate([
        edge_weight.astype(jnp.float32),
        jnp.ones((N_NODES,), jnp.float32),
        jnp.zeros((pad,), jnp.float32),
    ])
    row3 = row.reshape(NW, NCH, CHUNK)
    col3 = col.reshape(NW, NCH, CHUNK)
    w3 = w.reshape(NW, NCH, CHUNK)
    rowf = row.reshape(NW, E_PER_W)
    colf = col.reshape(NW, E_PER_W)
    wf = w.reshape(NW, E_PER_W)
    del row3

    degp = _deg_kernel(col3, w3)
    dinv = _dinv(degp).reshape(N_PAD)

    counts = _count_kernel(colf)
    off, meta = _offsets(counts)
    rowp, colp, wp = _part_kernel(rowf, colf, wf, off)
    normp = _norm_kernel(rowp, colp, wp, dinv)

    h0 = _mlp(x, W1, b1, W2, b2)
    h0p = jnp.concatenate(
        [h0, jnp.zeros((N_PAD - N_NODES, OUT_CH), jnp.float32)])

    h = h0p
    for _ in range(K):
        aggf = _round_kernel(rowp, colp, normp, meta, h)
        h = _combine(aggf.reshape(N_PAD, OUT_CH), h0p)
    return h[:N_NODES]


# speedup vs baseline: 2.3278x; 2.3278x over previous
import functools

import jax
import jax.numpy as jnp
from jax import lax
from jax.experimental import pallas as pl
from jax.experimental.pallas import tpu as pltpu
from jax.experimental.pallas import tpu_sc as plsc

N_NODES = 10000
N_PAD = 10240
IN_CH, HID_CH, OUT_CH = 128, 64, 64
K = 10
ALPHA = 0.1

NC, NS = 2, 16
NW = NC * NS
CHUNK = 128
NCH = 82
E_PER_W = NCH * CHUNK
E_PAD = NW * E_PER_W
ROWS_PER_TILE = N_NODES // NS

_mesh = plsc.VectorSubcoreMesh(
    core_axis_name="c", subcore_axis_name="s", num_cores=NC, num_subcores=NS)
_sc_params = pltpu.CompilerParams(
    needs_layout_passes=False, use_tc_tiling_on_sc=False)


@functools.partial(
    pl.kernel,
    out_type=jax.ShapeDtypeStruct((NC, N_PAD), jnp.float32),
    mesh=_mesh,
    compiler_params=_sc_params,
    scratch_types=[
        pltpu.VMEM((NCH, CHUNK), jnp.int32),
        pltpu.VMEM((NCH, CHUNK), jnp.float32),
        pltpu.VMEM((N_PAD,), jnp.float32),
        pltpu.VMEM_SHARED((N_PAD,), jnp.float32),
    ],
)
def _deg_kernel(col_hbm, w_hbm, degp_hbm, col_v, w_v, bounce_v, deg_sh):
    cid = lax.axis_index("c")
    sid = lax.axis_index("s")
    wid = sid * NC + cid
    pltpu.sync_copy(col_hbm.at[wid], col_v)
    pltpu.sync_copy(w_hbm.at[wid], w_v)

    zero16 = jnp.zeros((16,), jnp.float32)

    def _zero(i, carry):
        bounce_v[pl.ds(i * 16, 16)] = zero16
        return carry

    lax.fori_loop(0, N_PAD // 16, _zero, 0)

    @pl.when(sid == 0)
    def _():
        pltpu.sync_copy(bounce_v, deg_sh)

    plsc.subcore_barrier()

    def _scatter(j, carry):
        pltpu.sync_copy(w_v.at[j], deg_sh.at[col_v.at[j]], add=True)
        return carry

    lax.fori_loop(0, NCH, _scatter, 0)
    plsc.subcore_barrier()

    @pl.when(sid == 0)
    def _():
        pltpu.sync_copy(deg_sh, bounce_v)
        pltpu.sync_copy(bounce_v, degp_hbm.at[cid])


@functools.partial(
    pl.kernel,
    out_type=jax.ShapeDtypeStruct((NW, E_PER_W), jnp.float32),
    mesh=_mesh,
    compiler_params=_sc_params,
    scratch_types=[
        pltpu.VMEM((N_PAD,), jnp.float32),
        pltpu.VMEM((E_PER_W,), jnp.int32),
        pltpu.VMEM((E_PER_W,), jnp.int32),
        pltpu.VMEM((E_PER_W,), jnp.float32),
        pltpu.VMEM((E_PER_W,), jnp.float32),
    ],
)
def _norm_kernel(rowf, colf, wf, dinv_hbm, normf, dinv_v, row_v, col_v, w_v,
                 norm_v):
    cid = lax.axis_index("c")
    sid = lax.axis_index("s")
    wid = sid * NC + cid
    pltpu.sync_copy(dinv_hbm, dinv_v)
    pltpu.sync_copy(rowf.at[wid], row_v)
    pltpu.sync_copy(colf.at[wid], col_v)
    pltpu.sync_copy(wf.at[wid], w_v)

    def _body(g, carry):
        r16 = row_v[pl.ds(g * 16, 16)]
        c16 = col_v[pl.ds(g * 16, 16)]
        w16 = w_v[pl.ds(g * 16, 16)]
        dr = plsc.load_gather(dinv_v, [r16])
        dc = plsc.load_gather(dinv_v, [c16])
        norm_v[pl.ds(g * 16, 16)] = dr * w16 * dc
        return carry

    lax.fori_loop(0, E_PER_W // 16, _body, 0)
    pltpu.sync_copy(norm_v, normf.at[wid])


def _scale_chunk(buf, norm_v, j):
    for g in range(CHUNK // 16):
        n16 = norm_v[pl.ds(j * CHUNK + g * 16, 16)]
        for e in range(16):
            ne = jnp.broadcast_to(n16[e], (16,))
            r = g * 16 + e
            for f in range(OUT_CH // 16):
                buf[r, pl.ds(f * 16, 16)] = buf[r, pl.ds(f * 16, 16)] * ne


@functools.partial(
    pl.kernel,
    out_type=jax.ShapeDtypeStruct((2 * NC, N_NODES, OUT_CH), jnp.float32),
    mesh=_mesh,
    compiler_params=_sc_params,
    scratch_types=[
        pltpu.VMEM((NCH, CHUNK), jnp.int32),
        pltpu.VMEM((NCH, CHUNK), jnp.int32),
        pltpu.VMEM((E_PER_W,), jnp.float32),
        pltpu.VMEM((CHUNK, OUT_CH), jnp.float32),
        pltpu.VMEM((CHUNK, OUT_CH), jnp.float32),
        pltpu.VMEM_SHARED((N_NODES, OUT_CH), jnp.float32),
        pltpu.VMEM_SHARED((N_NODES, OUT_CH), jnp.float32),
        pltpu.SemaphoreType.DMA,
        pltpu.SemaphoreType.DMA,
        pltpu.SemaphoreType.DMA,
        pltpu.SemaphoreType.DMA,
    ],
)
def _round_kernel(row3, col3, normf, h_hbm, p_hbm, row_v, col_v, norm_v,
                  gb0, gb1, agg_sh, agg_sh2, gsem0, gsem1, ssem0, ssem1):
    cid = lax.axis_index("c")
    sid = lax.axis_index("s")
    wid = sid * NC + cid
    par0 = lax.rem(sid, 2) == 0

    def _scat_issue(gb, idxref, sem):
        @pl.when(par0)
        def _():
            pltpu.async_copy(gb, agg_sh.at[idxref], sem, add=True)

        @pl.when(jnp.logical_not(par0))
        def _():
            pltpu.async_copy(gb, agg_sh2.at[idxref], sem, add=True)

    def _scat_wait(gb, idxref, sem):
        @pl.when(par0)
        def _():
            pltpu.make_async_copy(gb, agg_sh.at[idxref], sem).wait()

        @pl.when(jnp.logical_not(par0))
        def _():
            pltpu.make_async_copy(gb, agg_sh2.at[idxref], sem).wait()
    pltpu.sync_copy(row3.at[wid], row_v)
    pltpu.sync_copy(col3.at[wid], col_v)
    pltpu.sync_copy(normf.at[wid], norm_v)

    zero16 = jnp.zeros((16,), jnp.float32)

    def _zero(i, carry):
        for f in range(OUT_CH // 16):
            gb0[i, pl.ds(f * 16, 16)] = zero16
        return carry

    lax.fori_loop(0, CHUNK, _zero, 0)
    base = sid * ROWS_PER_TILE
    rem = ROWS_PER_TILE % CHUNK
    for a_sh in (agg_sh, agg_sh2):
        for t in range(ROWS_PER_TILE // CHUNK):
            pltpu.sync_copy(gb0, a_sh.at[pl.ds(base + t * CHUNK, CHUNK)])
        if rem:
            pltpu.sync_copy(
                gb0.at[pl.ds(0, rem)],
                a_sh.at[pl.ds(base + (ROWS_PER_TILE // CHUNK) * CHUNK, rem)])
    plsc.subcore_barrier()

    pltpu.async_copy(h_hbm.at[row_v.at[0]], gb0, gsem0)
    pltpu.async_copy(h_hbm.at[row_v.at[1]], gb1, gsem1)

    def _iter(t, carry):
        j0 = 2 * t
        j1 = 2 * t + 1
        pltpu.make_async_copy(h_hbm.at[row_v.at[j0]], gb0, gsem0).wait()
        _scale_chunk(gb0, norm_v, j0)
        _scat_issue(gb0, col_v.at[j0], ssem0)
        pltpu.make_async_copy(h_hbm.at[row_v.at[j1]], gb1, gsem1).wait()
        _scale_chunk(gb1, norm_v, j1)
        _scat_issue(gb1, col_v.at[j1], ssem1)
        _scat_wait(gb0, col_v.at[j0], ssem0)

        @pl.when(j0 + 2 < NCH)
        def _():
            pltpu.async_copy(h_hbm.at[row_v.at[j0 + 2]], gb0, gsem0)

        _scat_wait(gb1, col_v.at[j1], ssem1)

        @pl.when(j1 + 2 < NCH)
        def _():
            pltpu.async_copy(h_hbm.at[row_v.at[j1 + 2]], gb1, gsem1)

        return carry

    lax.fori_loop(0, NCH // 2, _iter, 0)
    plsc.subcore_barrier()

    for pi, a_sh in ((2 * cid, agg_sh), (2 * cid + 1, agg_sh2)):
        for t in range(ROWS_PER_TILE // CHUNK):
            pltpu.sync_copy(a_sh.at[pl.ds(base + t * CHUNK, CHUNK)], gb0)
            pltpu.sync_copy(gb0, p_hbm.at[pi, pl.ds(base + t * CHUNK, CHUNK)])
        if rem:
            off = base + (ROWS_PER_TILE // CHUNK) * CHUNK
            pltpu.sync_copy(a_sh.at[pl.ds(off, rem)], gb0.at[pl.ds(0, rem)])
            pltpu.sync_copy(gb0.at[pl.ds(0, rem)],
                            p_hbm.at[pi, pl.ds(off, rem)])


def _mlp_body(x_ref, w1_ref, b1_ref, w2_ref, b2_ref, o_ref):
    h = jnp.dot(x_ref[...], w1_ref[...], preferred_element_type=jnp.float32)
    h = jnp.maximum(h + b1_ref[...], 0.0)
    h = jnp.dot(h, w2_ref[...], preferred_element_type=jnp.float32)
    o_ref[...] = h + b2_ref[...]


def _mlp(x, W1, b1, W2, b2):
    blk = 1000
    return pl.pallas_call(
        _mlp_body,
        grid=(N_NODES // blk,),
        in_specs=[
            pl.BlockSpec((blk, IN_CH), lambda i: (i, 0)),
            pl.BlockSpec((IN_CH, HID_CH), lambda i: (0, 0)),
            pl.BlockSpec((1, HID_CH), lambda i: (0, 0)),
            pl.BlockSpec((HID_CH, OUT_CH), lambda i: (0, 0)),
            pl.BlockSpec((1, OUT_CH), lambda i: (0, 0)),
        ],
        out_specs=pl.BlockSpec((blk, OUT_CH), lambda i: (i, 0)),
        out_shape=jax.ShapeDtypeStruct((N_NODES, OUT_CH), jnp.float32),
    )(x, W1, b1.reshape(1, HID_CH), W2, b2.reshape(1, OUT_CH))


def _dinv_body(degp_ref, o_ref):
    deg = degp_ref[0] + degp_ref[1]
    safe = jnp.where(deg > 0, deg, 1.0)
    o_ref[...] = jnp.where(deg > 0, lax.rsqrt(safe), 0.0)


def _dinv(degp):
    return pl.pallas_call(
        _dinv_body,
        out_shape=jax.ShapeDtypeStruct((N_PAD // 128, 128), jnp.float32),
    )(degp.reshape(NC, N_PAD // 128, 128))


def _combine_body(p_ref, h0_ref, o_ref):
    agg = (p_ref[0] + p_ref[1]) + (p_ref[2] + p_ref[3])
    o_ref[...] = (1.0 - ALPHA) * agg + ALPHA * h0_ref[...]


def _combine(p, h0):
    blk = 1000
    return pl.pallas_call(
        _combine_body,
        grid=(N_NODES // blk,),
        in_specs=[
            pl.BlockSpec((2 * NC, blk, OUT_CH), lambda i: (0, i, 0)),
            pl.BlockSpec((blk, OUT_CH), lambda i: (i, 0)),
        ],
        out_specs=pl.BlockSpec((blk, OUT_CH), lambda i: (i, 0)),
        out_shape=jax.ShapeDtypeStruct((N_NODES, OUT_CH), jnp.float32),
    )(p, h0)


def kernel(x, edge_index, edge_weight, W1, b1, W2, b2):
    pad = E_PAD - (edge_index.shape[1] + N_NODES)
    loop = jnp.arange(N_NODES, dtype=jnp.int32)
    zpad_i = jnp.zeros((pad,), jnp.int32)
    row = jnp.concatenate([edge_index[0].astype(jnp.int32), loop, zpad_i])
    col = jnp.concatenate([edge_index[1].astype(jnp.int32), loop, zpad_i])
    w = jnp.concatenate([
        edge_weight.astype(jnp.float32),
        jnp.ones((N_NODES,), jnp.float32),
        jnp.zeros((pad,), jnp.float32),
    ])
    row3 = row.reshape(NW, NCH, CHUNK)
    col3 = col.reshape(NW, NCH, CHUNK)
    w3 = w.reshape(NW, NCH, CHUNK)
    rowf = row.reshape(NW, E_PER_W)
    colf = col.reshape(NW, E_PER_W)
    wf = w.reshape(NW, E_PER_W)

    degp = _deg_kernel(col3, w3)
    dinv = _dinv(degp).reshape(N_PAD)
    normf = _norm_kernel(rowf, colf, wf, dinv)
    h0 = _mlp(x, W1, b1, W2, b2)

    h = h0
    for _ in range(K):
        p = _round_kernel(row3, col3, normf, h)
        h = _combine(p, h0)
    return h


# speedup vs baseline: 2.3908x; 1.0270x over previous
import functools

import jax
import jax.numpy as jnp
from jax import lax
from jax.experimental import pallas as pl
from jax.experimental.pallas import tpu as pltpu
from jax.experimental.pallas import tpu_sc as plsc

N_NODES = 10000
N_PAD = 10240
IN_CH, HID_CH, OUT_CH = 128, 64, 64
K = 10
ALPHA = 0.1

NC, NS = 2, 16
NW = NC * NS
CHUNK = 128
NCH = 82
E_PER_W = NCH * CHUNK
E_PAD = NW * E_PER_W
ROWS_PER_TILE = N_NODES // NS

_mesh = plsc.VectorSubcoreMesh(
    core_axis_name="c", subcore_axis_name="s", num_cores=NC, num_subcores=NS)
_sc_params = pltpu.CompilerParams(
    needs_layout_passes=False, use_tc_tiling_on_sc=False)


@functools.partial(
    pl.kernel,
    out_type=jax.ShapeDtypeStruct((NC, N_PAD), jnp.float32),
    mesh=_mesh,
    compiler_params=_sc_params,
    scratch_types=[
        pltpu.VMEM((NCH, CHUNK), jnp.int32),
        pltpu.VMEM((NCH, CHUNK), jnp.float32),
        pltpu.VMEM((N_PAD,), jnp.float32),
        pltpu.VMEM_SHARED((N_PAD,), jnp.float32),
    ],
)
def _deg_kernel(col_hbm, w_hbm, degp_hbm, col_v, w_v, bounce_v, deg_sh):
    cid = lax.axis_index("c")
    sid = lax.axis_index("s")
    wid = sid * NC + cid
    pltpu.sync_copy(col_hbm.at[wid], col_v)
    pltpu.sync_copy(w_hbm.at[wid], w_v)

    zero16 = jnp.zeros((16,), jnp.float32)

    def _zero(i, carry):
        bounce_v[pl.ds(i * 16, 16)] = zero16
        return carry

    lax.fori_loop(0, N_PAD // 16, _zero, 0)

    @pl.when(sid == 0)
    def _():
        pltpu.sync_copy(bounce_v, deg_sh)

    plsc.subcore_barrier()

    def _scatter(j, carry):
        pltpu.sync_copy(w_v.at[j], deg_sh.at[col_v.at[j]], add=True)
        return carry

    lax.fori_loop(0, NCH, _scatter, 0)
    plsc.subcore_barrier()

    @pl.when(sid == 0)
    def _():
        pltpu.sync_copy(deg_sh, bounce_v)
        pltpu.sync_copy(bounce_v, degp_hbm.at[cid])


@functools.partial(
    pl.kernel,
    out_type=jax.ShapeDtypeStruct((NW, E_PER_W), jnp.float32),
    mesh=_mesh,
    compiler_params=_sc_params,
    scratch_types=[
        pltpu.VMEM((N_PAD,), jnp.float32),
        pltpu.VMEM((E_PER_W,), jnp.int32),
        pltpu.VMEM((E_PER_W,), jnp.int32),
        pltpu.VMEM((E_PER_W,), jnp.float32),
        pltpu.VMEM((E_PER_W,), jnp.float32),
    ],
)
def _norm_kernel(rowf, colf, wf, dinv_hbm, normf, dinv_v, row_v, col_v, w_v,
                 norm_v):
    cid = lax.axis_index("c")
    sid = lax.axis_index("s")
    wid = sid * NC + cid
    pltpu.sync_copy(dinv_hbm, dinv_v)
    pltpu.sync_copy(rowf.at[wid], row_v)
    pltpu.sync_copy(colf.at[wid], col_v)
    pltpu.sync_copy(wf.at[wid], w_v)

    def _body(g, carry):
        r16 = row_v[pl.ds(g * 16, 16)]
        c16 = col_v[pl.ds(g * 16, 16)]
        w16 = w_v[pl.ds(g * 16, 16)]
        dr = plsc.load_gather(dinv_v, [r16])
        dc = plsc.load_gather(dinv_v, [c16])
        norm_v[pl.ds(g * 16, 16)] = dr * w16 * dc
        return carry

    lax.fori_loop(0, E_PER_W // 16, _body, 0)
    pltpu.sync_copy(norm_v, normf.at[wid])


def _scale_chunk(buf, norm_v, j):
    for g in range(CHUNK // 16):
        n16 = norm_v[pl.ds(j * CHUNK + g * 16, 16)]
        for e in range(16):
            ne = jnp.broadcast_to(n16[e], (16,))
            r = g * 16 + e
            for f in range(OUT_CH // 16):
                buf[r, pl.ds(f * 16, 16)] = buf[r, pl.ds(f * 16, 16)] * ne


@functools.partial(
    pl.kernel,
    out_type=jax.ShapeDtypeStruct((NC, N_NODES, OUT_CH), jnp.float32),
    mesh=_mesh,
    compiler_params=_sc_params,
    scratch_types=[
        pltpu.VMEM((NCH, CHUNK), jnp.int32),
        pltpu.VMEM((NCH, CHUNK), jnp.int32),
        pltpu.VMEM((E_PER_W,), jnp.float32),
        pltpu.VMEM((CHUNK, OUT_CH), jnp.float32),
        pltpu.VMEM((CHUNK, OUT_CH), jnp.float32),
        pltpu.VMEM_SHARED((N_NODES, OUT_CH), jnp.float32),
        pltpu.SemaphoreType.DMA,
        pltpu.SemaphoreType.DMA,
        pltpu.SemaphoreType.DMA,
        pltpu.SemaphoreType.DMA,
    ],
)
def _round_kernel(row3, col3, normf, h_hbm, p_hbm, row_v, col_v, norm_v,
                  gb0, gb1, agg_sh, gsem0, gsem1, ssem0, ssem1):
    cid = lax.axis_index("c")
    sid = lax.axis_index("s")
    wid = sid * NC + cid
    pltpu.sync_copy(row3.at[wid], row_v)
    pltpu.sync_copy(col3.at[wid], col_v)
    pltpu.sync_copy(normf.at[wid], norm_v)

    zero16 = jnp.zeros((16,), jnp.float32)

    def _zero(i, carry):
        for f in range(OUT_CH // 16):
            gb0[i, pl.ds(f * 16, 16)] = zero16
        return carry

    lax.fori_loop(0, CHUNK, _zero, 0)
    base = sid * ROWS_PER_TILE
    for t in range(ROWS_PER_TILE // CHUNK):
        pltpu.sync_copy(gb0, agg_sh.at[pl.ds(base + t * CHUNK, CHUNK)])
    rem = ROWS_PER_TILE % CHUNK
    if rem:
        pltpu.sync_copy(
            gb0.at[pl.ds(0, rem)],
            agg_sh.at[pl.ds(base + (ROWS_PER_TILE // CHUNK) * CHUNK, rem)])
    plsc.subcore_barrier()

    pltpu.async_copy(h_hbm.at[row_v.at[0]], gb0, gsem0)
    pltpu.async_copy(h_hbm.at[row_v.at[1]], gb1, gsem1)

    def _iter(t, carry):
        j0 = 2 * t
        j1 = 2 * t + 1
        pltpu.make_async_copy(h_hbm.at[row_v.at[j0]], gb0, gsem0).wait()
        _scale_chunk(gb0, norm_v, j0)
        pltpu.async_copy(gb0, agg_sh.at[col_v.at[j0]], ssem0, add=True)
        pltpu.make_async_copy(h_hbm.at[row_v.at[j1]], gb1, gsem1).wait()
        _scale_chunk(gb1, norm_v, j1)
        pltpu.async_copy(gb1, agg_sh.at[col_v.at[j1]], ssem1, add=True)
        pltpu.make_async_copy(gb0, agg_sh.at[col_v.at[j0]], ssem0).wait()

        @pl.when(j0 + 2 < NCH)
        def _():
            pltpu.async_copy(h_hbm.at[row_v.at[j0 + 2]], gb0, gsem0)

        pltpu.make_async_copy(gb1, agg_sh.at[col_v.at[j1]], ssem1).wait()

        @pl.when(j1 + 2 < NCH)
        def _():
            pltpu.async_copy(h_hbm.at[row_v.at[j1 + 2]], gb1, gsem1)

        return carry

    lax.fori_loop(0, NCH // 2, _iter, 0)
    plsc.subcore_barrier()

    for t in range(ROWS_PER_TILE // CHUNK):
        pltpu.sync_copy(agg_sh.at[pl.ds(base + t * CHUNK, CHUNK)], gb0)
        pltpu.sync_copy(gb0, p_hbm.at[cid, pl.ds(base + t * CHUNK, CHUNK)])
    if rem:
        off = base + (ROWS_PER_TILE // CHUNK) * CHUNK
        pltpu.sync_copy(agg_sh.at[pl.ds(off, rem)], gb0.at[pl.ds(0, rem)])
        pltpu.sync_copy(gb0.at[pl.ds(0, rem)], p_hbm.at[cid, pl.ds(off, rem)])


def _mlp_body(x_ref, w1_ref, b1_ref, w2_ref, b2_ref, o_ref):
    h = jnp.dot(x_ref[...], w1_ref[...], preferred_element_type=jnp.float32)
    h = jnp.maximum(h + b1_ref[...], 0.0)
    h = jnp.dot(h, w2_ref[...], preferred_element_type=jnp.float32)
    o_ref[...] = h + b2_ref[...]


def _mlp(x, W1, b1, W2, b2):
    blk = 1000
    return pl.pallas_call(
        _mlp_body,
        grid=(N_NODES // blk,),
        in_specs=[
            pl.BlockSpec((blk, IN_CH), lambda i: (i, 0)),
            pl.BlockSpec((IN_CH, HID_CH), lambda i: (0, 0)),
            pl.BlockSpec((1, HID_CH), lambda i: (0, 0)),
            pl.BlockSpec((HID_CH, OUT_CH), lambda i: (0, 0)),
            pl.BlockSpec((1, OUT_CH), lambda i: (0, 0)),
        ],
        out_specs=pl.BlockSpec((blk, OUT_CH), lambda i: (i, 0)),
        out_shape=jax.ShapeDtypeStruct((N_NODES, OUT_CH), jnp.float32),
    )(x, W1, b1.reshape(1, HID_CH), W2, b2.reshape(1, OUT_CH))


def _dinv_body(degp_ref, o_ref):
    deg = degp_ref[0] + degp_ref[1]
    safe = jnp.where(deg > 0, deg, 1.0)
    o_ref[...] = jnp.where(deg > 0, lax.rsqrt(safe), 0.0)


def _dinv(degp):
    return pl.pallas_call(
        _dinv_body,
        out_shape=jax.ShapeDtypeStruct((N_PAD // 128, 128), jnp.float32),
    )(degp.reshape(NC, N_PAD // 128, 128))


def _combine_body(p_ref, h0_ref, o_ref):
    agg = p_ref[0] + p_ref[1]
    o_ref[...] = (1.0 - ALPHA) * agg + ALPHA * h0_ref[...]


def _combine(p, h0):
    blk = 1000
    return pl.pallas_call(
        _combine_body,
        grid=(N_NODES // blk,),
        in_specs=[
            pl.BlockSpec((NC, blk, OUT_CH), lambda i: (0, i, 0)),
            pl.BlockSpec((blk, OUT_CH), lambda i: (i, 0)),
        ],
        out_specs=pl.BlockSpec((blk, OUT_CH), lambda i: (i, 0)),
        out_shape=jax.ShapeDtypeStruct((N_NODES, OUT_CH), jnp.float32),
    )(p, h0)


def kernel(x, edge_index, edge_weight, W1, b1, W2, b2):
    pad = E_PAD - (edge_index.shape[1] + N_NODES)
    loop = jnp.arange(N_NODES, dtype=jnp.int32)
    zpad_i = jnp.zeros((pad,), jnp.int32)
    row = jnp.concatenate([edge_index[0].astype(jnp.int32), loop, zpad_i])
    col = jnp.concatenate([edge_index[1].astype(jnp.int32), loop, zpad_i])
    w = jnp.concatenate([
        edge_weight.astype(jnp.float32),
        jnp.ones((N_NODES,), jnp.float32),
        jnp.zeros((pad,), jnp.float32),
    ])
    row3 = row.reshape(NW, NCH, CHUNK)
    col3 = col.reshape(NW, NCH, CHUNK)
    w3 = w.reshape(NW, NCH, CHUNK)
    rowf = row.reshape(NW, E_PER_W)
    colf = col.reshape(NW, E_PER_W)
    wf = w.reshape(NW, E_PER_W)

    degp = _deg_kernel(col3, w3)
    dinv = _dinv(degp).reshape(N_PAD)
    normf = _norm_kernel(rowf, colf, wf, dinv)
    h0 = _mlp(x, W1, b1, W2, b2)

    h = h0
    for _ in range(K):
        p = _round_kernel(row3, col3, normf, h)
        h = _combine(p, h0)
    return h
